# baseline (speedup 1.0000x reference)
import functools

import jax
import jax.numpy as jnp
from jax import lax
from jax.experimental import pallas as pl
from jax.experimental.pallas import tpu as pltpu
from jax.experimental.pallas import tpu_sc as plsc

_N = 10000
_E = 320000
_D = 128

_NC = 2
_NS = 16
_NW = _NC * _NS
_L = 16

_K = 128
_NPAD = 10240
_EPAD = _NW * 80 * _K
_NB = _EPAD // (_NW * _K)
_PH = 2
_NBP = _NB // _PH
_RPT = _NPAD // _NS

_mesh = plsc.VectorSubcoreMesh(core_axis_name="c", subcore_axis_name="s")


def _count_body(dst3d, cnt0, cnt1, dstv, onesv, zbuf, acc):
    cid = lax.axis_index("c")
    sid = lax.axis_index("s")
    wid = cid * _NS + sid
    zero16 = jnp.zeros((_L,), jnp.float32)
    one16 = jnp.ones((_L,), jnp.float32)

    pltpu.sync_copy(dst3d.at[:, wid], dstv)

    @pl.loop(0, _RPT // _L)
    def _(i):
        zbuf[pl.ds(i * _L, _L)] = zero16

    @pl.loop(0, _K // _L)
    def _(j):
        onesv[pl.ds(j * _L, _L)] = one16

    pltpu.sync_copy(zbuf, acc.at[pl.ds(sid * _RPT, _RPT)])
    plsc.subcore_barrier()

    @pl.loop(0, _NB)
    def _(j):
        pltpu.sync_copy(onesv, acc.at[dstv.at[j]], add=True)

    plsc.subcore_barrier()

    @pl.when(cid == 0)
    def _():
        pltpu.sync_copy(acc.at[pl.ds(sid * _RPT, _RPT)],
                        cnt0.at[pl.ds(sid * _RPT, _RPT)])

    @pl.when(cid == 1)
    def _():
        pltpu.sync_copy(acc.at[pl.ds(sid * _RPT, _RPT)],
                        cnt1.at[pl.ds(sid * _RPT, _RPT)])


_count_call = functools.partial(
    pl.kernel,
    out_type=(
        jax.ShapeDtypeStruct((_NPAD,), jnp.float32),
        jax.ShapeDtypeStruct((_NPAD,), jnp.float32),
    ),
    mesh=_mesh,
    scratch_types=[
        pltpu.VMEM((_NB, _K), jnp.int32),
        pltpu.VMEM((_K,), jnp.float32),
        pltpu.VMEM((_RPT,), jnp.float32),
        pltpu.VMEM_SHARED((_NPAD,), jnp.float32),
    ],
)(_count_body)


def _gs_body(hd, src3d, dst3d, out0, out1, srcv, dstv, buf0, buf1, acc, s0, s1):
    bufs = (buf0, buf1)
    sems = (s0, s1)
    cid = lax.axis_index("c")
    sid = lax.axis_index("s")
    wid = cid * _NS + sid
    zero16 = jnp.zeros((_L,), jnp.float32)

    @pl.loop(0, _K)
    def _(i):
        for c in range(_D // _L):
            buf0[i, pl.ds(c * _L, _L)] = zero16

    for r in range(_RPT // _K):
        pltpu.sync_copy(buf0, acc.at[pl.ds((sid * (_RPT // _K) + r) * _K, _K)])

    plsc.subcore_barrier()

    for phase in range(_PH):
        base = phase * _NBP
        pltpu.sync_copy(src3d.at[pl.ds(base, _NBP), wid], srcv)
        pltpu.sync_copy(dst3d.at[pl.ds(base, _NBP), wid], dstv)

        for b in range(2):
            pltpu.async_copy(hd.at[srcv.at[b]], bufs[b], sems[b])

        @pl.loop(0, _NBP // 2)
        def _(g):
            r0 = g * 2
            for b in range(2):
                r = r0 + b
                pltpu.make_async_copy(hd.at[srcv.at[r]], bufs[b], sems[b]).wait()
                pltpu.sync_copy(bufs[b], acc.at[dstv.at[r]], add=True)

                @pl.when(r + 2 < _NBP)
                def _():
                    pltpu.async_copy(hd.at[srcv.at[r + 2]], bufs[b], sems[b])

    plsc.subcore_barrier()

    @pl.when(cid == 0)
    def _():
        pltpu.sync_copy(acc.at[pl.ds(sid * _RPT, _RPT)],
                        out0.at[pl.ds(sid * _RPT, _RPT)])

    @pl.when(cid == 1)
    def _():
        pltpu.sync_copy(acc.at[pl.ds(sid * _RPT, _RPT)],
                        out1.at[pl.ds(sid * _RPT, _RPT)])


_gs_call = functools.partial(
    pl.kernel,
    out_type=(
        jax.ShapeDtypeStruct((_NPAD, _D), jnp.float32),
        jax.ShapeDtypeStruct((_NPAD, _D), jnp.float32),
    ),
    mesh=_mesh,
    scratch_types=[
        pltpu.VMEM((_NBP, _K), jnp.int32),
        pltpu.VMEM((_NBP, _K), jnp.int32),
        pltpu.VMEM((_K, _D), jnp.float32),
        pltpu.VMEM((_K, _D), jnp.float32),
        pltpu.VMEM_SHARED((_NPAD, _D), jnp.float32),
        pltpu.SemaphoreType.DMA,
        pltpu.SemaphoreType.DMA,
    ],
)(_gs_body)


_BLK = 2000


def _tc1_body(x_ref, w_ref, c0_ref, c1_ref, hd_ref):
    d = lax.rsqrt(1.0 + c0_ref[...] + c1_ref[...])
    hd_ref[...] = jnp.dot(x_ref[...], w_ref[...],
                          preferred_element_type=jnp.float32) * d


def _tc1(x, W1, c0, c1):
    return pl.pallas_call(
        _tc1_body,
        grid=(_N // _BLK,),
        in_specs=[
            pl.BlockSpec((_BLK, _D), lambda i: (i, 0)),
            pl.BlockSpec((_D, _D), lambda i: (0, 0)),
            pl.BlockSpec((_BLK, 1), lambda i: (i, 0)),
            pl.BlockSpec((_BLK, 1), lambda i: (i, 0)),
        ],
        out_specs=pl.BlockSpec((_BLK, _D), lambda i: (i, 0)),
        out_shape=jax.ShapeDtypeStruct((_N, _D), jnp.float32),
    )(x, W1, c0, c1)


def _tc2_body(a0_ref, a1_ref, hd1_ref, c0_ref, c1_ref, w_ref, b1_ref, hd2_ref):
    d = lax.rsqrt(1.0 + c0_ref[...] + c1_ref[...])
    s = a0_ref[...] + a1_ref[...] + hd1_ref[...]
    h1 = jnp.maximum(d * s + b1_ref[...], 0.0)
    hd2_ref[...] = jnp.dot(h1, w_ref[...],
                           preferred_element_type=jnp.float32) * d


def _tc2(a0, a1, hd1, c0, c1, W2, b1):
    return pl.pallas_call(
        _tc2_body,
        grid=(_N // _BLK,),
        in_specs=[
            pl.BlockSpec((_BLK, _D), lambda i: (i, 0)),
            pl.BlockSpec((_BLK, _D), lambda i: (i, 0)),
            pl.BlockSpec((_BLK, _D), lambda i: (i, 0)),
            pl.BlockSpec((_BLK, 1), lambda i: (i, 0)),
            pl.BlockSpec((_BLK, 1), lambda i: (i, 0)),
            pl.BlockSpec((_D, _D), lambda i: (0, 0)),
            pl.BlockSpec((1, _D), lambda i: (0, 0)),
        ],
        out_specs=pl.BlockSpec((_BLK, _D), lambda i: (i, 0)),
        out_shape=jax.ShapeDtypeStruct((_N, _D), jnp.float32),
    )(a0, a1, hd1, c0, c1, W2, b1)


def _tc3_body(a0_ref, a1_ref, hd2_ref, c0_ref, c1_ref, b2_ref, out_ref):
    d = lax.rsqrt(1.0 + c0_ref[...] + c1_ref[...])
    out_ref[...] = d * (a0_ref[...] + a1_ref[...] + hd2_ref[...]) + b2_ref[...]


def _tc3(a0, a1, hd2, c0, c1, b2):
    return pl.pallas_call(
        _tc3_body,
        grid=(_N // _BLK,),
        in_specs=[
            pl.BlockSpec((_BLK, _D), lambda i: (i, 0)),
            pl.BlockSpec((_BLK, _D), lambda i: (i, 0)),
            pl.BlockSpec((_BLK, _D), lambda i: (i, 0)),
            pl.BlockSpec((_BLK, 1), lambda i: (i, 0)),
            pl.BlockSpec((_BLK, 1), lambda i: (i, 0)),
            pl.BlockSpec((1, _D), lambda i: (0, 0)),
        ],
        out_specs=pl.BlockSpec((_BLK, _D), lambda i: (i, 0)),
        out_shape=jax.ShapeDtypeStruct((_N, _D), jnp.float32),
    )(a0, a1, hd2, c0, c1, b2)


def kernel(x, edge_index, W1, b1, W2, b2):
    npad = _EPAD - _E
    pad_dst = _N + (jnp.arange(npad, dtype=jnp.int32) % (_NPAD - _N))
    pad_src = jnp.arange(npad, dtype=jnp.int32) % _N
    src3d = jnp.concatenate(
        [edge_index[0], pad_src]).reshape(_NB, _NW, _K)
    dst3d = jnp.concatenate(
        [edge_index[1], pad_dst]).reshape(_NB, _NW, _K)

    c0p, c1p = _count_call(dst3d)
    c0 = c0p.reshape(_NPAD, 1)
    c1 = c1p.reshape(_NPAD, 1)

    b1r = b1.reshape(1, _D)
    b2r = b2.reshape(1, _D)

    hd1 = _tc1(x, W1, c0, c1)
    a0, a1 = _gs_call(hd1, src3d, dst3d)
    hd2 = _tc2(a0, a1, hd1, c0, c1, W2, b1r)
    g0, g1 = _gs_call(hd2, src3d, dst3d)
    return _tc3(g0, g1, hd2, c0, c1, b2r)


# speedup vs baseline: 1.0316x; 1.0316x over previous
import functools

import jax
import jax.numpy as jnp
from jax import lax
from jax.experimental import pallas as pl
from jax.experimental.pallas import tpu as pltpu
from jax.experimental.pallas import tpu_sc as plsc

_N = 10000
_E = 320000
_D = 128

_NC = 2
_NS = 16
_NW = _NC * _NS
_L = 16

_K = 128
_NPAD = 10240
_EPAD = _NW * 80 * _K
_NB = _EPAD // (_NW * _K)
_PH = 2
_NBP = _NB // _PH
_RPT = _NPAD // _NS

_mesh = plsc.VectorSubcoreMesh(core_axis_name="c", subcore_axis_name="s")


def _count_body(ei4d, cnt0, cnt1, dstv, onesv, zbuf, acc):
    cid = lax.axis_index("c")
    sid = lax.axis_index("s")
    wid = cid * _NS + sid
    zero16 = jnp.zeros((_L,), jnp.float32)
    one16 = jnp.ones((_L,), jnp.float32)

    pltpu.sync_copy(ei4d.at[1, :, wid], dstv)

    @pl.loop(0, _RPT // _L)
    def _(i):
        zbuf[pl.ds(i * _L, _L)] = zero16

    @pl.loop(0, _K // _L)
    def _(j):
        onesv[pl.ds(j * _L, _L)] = one16

    pltpu.sync_copy(zbuf, acc.at[pl.ds(sid * _RPT, _RPT)])
    plsc.subcore_barrier()

    @pl.loop(0, _NB)
    def _(j):
        pltpu.sync_copy(onesv, acc.at[dstv.at[j]], add=True)

    plsc.subcore_barrier()

    @pl.when(cid == 0)
    def _():
        pltpu.sync_copy(acc.at[pl.ds(sid * _RPT, _RPT)],
                        cnt0.at[pl.ds(sid * _RPT, _RPT)])

    @pl.when(cid == 1)
    def _():
        pltpu.sync_copy(acc.at[pl.ds(sid * _RPT, _RPT)],
                        cnt1.at[pl.ds(sid * _RPT, _RPT)])


_count_call = functools.partial(
    pl.kernel,
    out_type=(
        jax.ShapeDtypeStruct((_NPAD,), jnp.float32),
        jax.ShapeDtypeStruct((_NPAD,), jnp.float32),
    ),
    mesh=_mesh,
    scratch_types=[
        pltpu.VMEM((_NB, _K), jnp.int32),
        pltpu.VMEM((_K,), jnp.float32),
        pltpu.VMEM((_RPT,), jnp.float32),
        pltpu.VMEM_SHARED((_NPAD,), jnp.float32),
    ],
)(_count_body)


def _gs_body(hd, ei4d, out0, out1, srcv, dstv, buf0, buf1, acc, s0, s1):
    bufs = (buf0, buf1)
    sems = (s0, s1)
    cid = lax.axis_index("c")
    sid = lax.axis_index("s")
    wid = cid * _NS + sid
    zero16 = jnp.zeros((_L,), jnp.float32)

    @pl.loop(0, _K)
    def _(i):
        for c in range(_D // _L):
            buf0[i, pl.ds(c * _L, _L)] = zero16

    for r in range(_RPT // _K):
        pltpu.sync_copy(buf0, acc.at[pl.ds((sid * (_RPT // _K) + r) * _K, _K)])

    plsc.subcore_barrier()

    for phase in range(_PH):
        base = phase * _NBP
        pltpu.sync_copy(ei4d.at[0, pl.ds(base, _NBP), wid], srcv)
        pltpu.sync_copy(ei4d.at[1, pl.ds(base, _NBP), wid], dstv)

        for b in range(2):
            pltpu.async_copy(hd.at[srcv.at[b]], bufs[b], sems[b])

        @pl.loop(0, _NBP // 2)
        def _(g):
            r0 = g * 2
            for b in range(2):
                r = r0 + b
                pltpu.make_async_copy(hd.at[srcv.at[r]], bufs[b], sems[b]).wait()
                pltpu.sync_copy(bufs[b], acc.at[dstv.at[r]], add=True)

                @pl.when(r + 2 < _NBP)
                def _():
                    pltpu.async_copy(hd.at[srcv.at[r + 2]], bufs[b], sems[b])

    plsc.subcore_barrier()

    @pl.when(cid == 0)
    def _():
        pltpu.sync_copy(acc.at[pl.ds(sid * _RPT, _RPT)],
                        out0.at[pl.ds(sid * _RPT, _RPT)])

    @pl.when(cid == 1)
    def _():
        pltpu.sync_copy(acc.at[pl.ds(sid * _RPT, _RPT)],
                        out1.at[pl.ds(sid * _RPT, _RPT)])


_gs_call = functools.partial(
    pl.kernel,
    out_type=(
        jax.ShapeDtypeStruct((_NPAD, _D), jnp.float32),
        jax.ShapeDtypeStruct((_NPAD, _D), jnp.float32),
    ),
    mesh=_mesh,
    scratch_types=[
        pltpu.VMEM((_NBP, _K), jnp.int32),
        pltpu.VMEM((_NBP, _K), jnp.int32),
        pltpu.VMEM((_K, _D), jnp.float32),
        pltpu.VMEM((_K, _D), jnp.float32),
        pltpu.VMEM_SHARED((_NPAD, _D), jnp.float32),
        pltpu.SemaphoreType.DMA,
        pltpu.SemaphoreType.DMA,
    ],
)(_gs_body)


_BLK = 2000


def _tc1_body(x_ref, w_ref, c0_ref, c1_ref, hd_ref):
    d = lax.rsqrt(1.0 + c0_ref[...] + c1_ref[...])
    hd_ref[...] = jnp.dot(x_ref[...], w_ref[...],
                          preferred_element_type=jnp.float32) * d


def _tc1(x, W1, c0, c1):
    return pl.pallas_call(
        _tc1_body,
        grid=(_N // _BLK,),
        in_specs=[
            pl.BlockSpec((_BLK, _D), lambda i: (i, 0)),
            pl.BlockSpec((_D, _D), lambda i: (0, 0)),
            pl.BlockSpec((_BLK, 1), lambda i: (i, 0)),
            pl.BlockSpec((_BLK, 1), lambda i: (i, 0)),
        ],
        out_specs=pl.BlockSpec((_BLK, _D), lambda i: (i, 0)),
        out_shape=jax.ShapeDtypeStruct((_N, _D), jnp.float32),
    )(x, W1, c0, c1)


def _tc2_body(a0_ref, a1_ref, hd1_ref, c0_ref, c1_ref, w_ref, b1_ref, hd2_ref):
    d = lax.rsqrt(1.0 + c0_ref[...] + c1_ref[...])
    s = a0_ref[...] + a1_ref[...] + hd1_ref[...]
    h1 = jnp.maximum(d * s + b1_ref[...], 0.0)
    hd2_ref[...] = jnp.dot(h1, w_ref[...],
                           preferred_element_type=jnp.float32) * d


def _tc2(a0, a1, hd1, c0, c1, W2, b1):
    return pl.pallas_call(
        _tc2_body,
        grid=(_N // _BLK,),
        in_specs=[
            pl.BlockSpec((_BLK, _D), lambda i: (i, 0)),
            pl.BlockSpec((_BLK, _D), lambda i: (i, 0)),
            pl.BlockSpec((_BLK, _D), lambda i: (i, 0)),
            pl.BlockSpec((_BLK, 1), lambda i: (i, 0)),
            pl.BlockSpec((_BLK, 1), lambda i: (i, 0)),
            pl.BlockSpec((_D, _D), lambda i: (0, 0)),
            pl.BlockSpec((1, _D), lambda i: (0, 0)),
        ],
        out_specs=pl.BlockSpec((_BLK, _D), lambda i: (i, 0)),
        out_shape=jax.ShapeDtypeStruct((_N, _D), jnp.float32),
    )(a0, a1, hd1, c0, c1, W2, b1)


def _tc3_body(a0_ref, a1_ref, hd2_ref, c0_ref, c1_ref, b2_ref, out_ref):
    d = lax.rsqrt(1.0 + c0_ref[...] + c1_ref[...])
    out_ref[...] = d * (a0_ref[...] + a1_ref[...] + hd2_ref[...]) + b2_ref[...]


def _tc3(a0, a1, hd2, c0, c1, b2):
    return pl.pallas_call(
        _tc3_body,
        grid=(_N // _BLK,),
        in_specs=[
            pl.BlockSpec((_BLK, _D), lambda i: (i, 0)),
            pl.BlockSpec((_BLK, _D), lambda i: (i, 0)),
            pl.BlockSpec((_BLK, _D), lambda i: (i, 0)),
            pl.BlockSpec((_BLK, 1), lambda i: (i, 0)),
            pl.BlockSpec((_BLK, 1), lambda i: (i, 0)),
            pl.BlockSpec((1, _D), lambda i: (0, 0)),
        ],
        out_specs=pl.BlockSpec((_BLK, _D), lambda i: (i, 0)),
        out_shape=jax.ShapeDtypeStruct((_N, _D), jnp.float32),
    )(a0, a1, hd2, c0, c1, b2)


def kernel(x, edge_index, W1, b1, W2, b2):
    npad = _EPAD - _E
    pad_dst = _N + (jnp.arange(npad, dtype=jnp.int32) % (_NPAD - _N))
    pad_src = jnp.arange(npad, dtype=jnp.int32) % _N
    ei4d = jnp.concatenate(
        [edge_index, jnp.stack([pad_src, pad_dst])], axis=1
    ).reshape(2, _NB, _NW, _K)

    c0p, c1p = _count_call(ei4d)
    c0 = c0p.reshape(_NPAD, 1)
    c1 = c1p.reshape(_NPAD, 1)

    b1r = b1.reshape(1, _D)
    b2r = b2.reshape(1, _D)

    hd1 = _tc1(x, W1, c0, c1)
    a0, a1 = _gs_call(hd1, ei4d)
    hd2 = _tc2(a0, a1, hd1, c0, c1, W2, b1r)
    g0, g1 = _gs_call(hd2, ei4d)
    return _tc3(g0, g1, hd2, c0, c1, b2r)
